# baseline (speedup 1.0000x reference)
import functools

import jax
import jax.numpy as jnp
from jax import lax
from jax.experimental import pallas as pl
from jax.experimental.pallas import tpu as pltpu
from jax.experimental.pallas import tpu_sc as plsc

MARGIN = 0.5
NUM_ANCHORS = 2048
D = 256
NC = 2
NS = 16
NW = NC * NS
L = 16
DSL = D // L
DSL2 = D // (2 * L)


def _vsqrt(x):
    i = lax.bitcast_convert_type(x, jnp.int32)
    i = jnp.int32(0x5F3759DF) - lax.shift_right_arithmetic(i, 1)
    y = lax.bitcast_convert_type(i, jnp.float32)
    for _ in range(3):
        y = y * (1.5 - 0.5 * x * y * y)
    return x * y


def _make_sc_kernel(total_tokens):
    tpw = total_tokens // NW
    chunk = 64
    nbuf = 4
    nch = tpw // chunk
    mesh = plsc.VectorSubcoreMesh(
        core_axis_name="c", subcore_axis_name="s",
        num_cores=NC, num_subcores=NS)

    @functools.partial(
        pl.kernel,
        out_type=jax.ShapeDtypeStruct((NW, 2 * L), jnp.float32),
        mesh=mesh,
        compiler_params=pltpu.CompilerParams(needs_layout_passes=False),
        scratch_types=[
            pltpu.VMEM((tpw,), jnp.int32),
            pltpu.VMEM((tpw,), jnp.int32),
            pltpu.VMEM((nbuf, chunk, D), jnp.float32),
            pltpu.VMEM((nbuf, chunk, D // 2), jnp.int32),
            pltpu.VMEM((L * L,), jnp.float32),
            pltpu.VMEM((2 * L,), jnp.float32),
        ] + [pltpu.SemaphoreType.DMA] * (2 * nbuf),
    )
    def sc_kernel(z_hbm, ids_hbm, cent_hbm, out_hbm,
                  idx_raw, idx_cl, z_v, cent_v, m_v, acc_v, *sems_flat):
        wid = lax.axis_index("c") * NS + lax.axis_index("s")
        base = wid * tpw

        pltpu.sync_copy(ids_hbm.at[pl.ds(base, tpw)], idx_raw)

        def clamp_body(j, _):
            v = idx_raw[pl.ds(j * L, L)]
            idx_cl[pl.ds(j * L, L)] = jnp.maximum(v, 0)
            return _
        lax.fori_loop(0, tpw // L, clamp_body, None)

        zero = jnp.zeros((L,), jnp.float32)
        acc_v[pl.ds(0, L)] = zero
        acc_v[pl.ds(L, L)] = zero

        sems = tuple((sems_flat[2 * b], sems_flat[2 * b + 1])
                     for b in range(nbuf))
        rows = lax.iota(jnp.int32, L)

        def copies(g, buf):
            sz, sc = sems[buf]
            return (
                pltpu.make_async_copy(
                    z_hbm.at[pl.ds(base + g * chunk, chunk)],
                    z_v.at[buf], sz),
                pltpu.make_async_copy(
                    cent_hbm.at[idx_cl.at[pl.ds(g * chunk, chunk)]],
                    cent_v.at[buf], sc),
            )

        def issue(g, buf):
            for cp in copies(g, buf):
                cp.start()

        def process(g, buf, last):
            tok0 = g * chunk
            for cp in copies(g, buf):
                cp.wait()
            zb = z_v.at[buf]
            cb = cent_v.at[buf]

            def grp_body(q, _):
                t0 = q * L
                for i in range(L):
                    t = t0 + i
                    acc = None
                    for k in range(DSL2):
                        zlo = zb[t, pl.ds(k * 2 * L, L)]
                        zhi = zb[t, pl.ds(k * 2 * L + L, L)]
                        cw = plsc.bitcast(cb[t, pl.ds(k * L, L)],
                                          jnp.bfloat16)
                        clo, chi = plsc.unpack(
                            cw, format=plsc.PackFormat.INTERLEAVED)
                        dlo = zlo - clo.astype(jnp.float32)
                        dhi = zhi - chi.astype(jnp.float32)
                        sq = dlo * dlo + dhi * dhi
                        acc = sq if acc is None else acc + sq
                    m_v[pl.ds(i * L, L)] = acc
                x = plsc.load_gather(m_v, [rows * L])
                for j in range(1, L):
                    x = x + plsc.load_gather(m_v, [rows * L + j])
                raw = idx_raw[pl.ds(tok0 + t0, L)]
                valid = raw >= 0
                h = jnp.maximum(_vsqrt(x) - MARGIN, 0.0)
                plsc.addupdate(acc_v.at[pl.ds(0, L)],
                               jnp.where(valid, h, 0.0))
                plsc.addupdate(acc_v.at[pl.ds(L, L)],
                               jnp.where(valid, 1.0, 0.0))
                return _
            lax.fori_loop(0, chunk // L, grp_body, None)

            @pl.when(jnp.logical_not(last))
            def _issue_next():
                issue(g + nbuf, buf)

        for b in range(nbuf):
            issue(b, b)

        def ring_body(p, _):
            last = p >= nch // nbuf - 1
            for b in range(nbuf):
                process(nbuf * p + b, b, last)
            return _
        lax.fori_loop(0, nch // nbuf, ring_body, None)

        pltpu.sync_copy(acc_v, out_hbm.at[wid])

    return sc_kernel


TC_BLK = 512


def _make_tc_kernel(tc_tokens, offset_blocks):
    grid = tc_tokens // TC_BLK

    def tc_body(z_ref, ids_ref, cent_ref, out_ref):
        ids = ids_ref[...]
        valid = ids >= 0
        idc = jnp.maximum(ids, 0)
        iota = lax.broadcasted_iota(jnp.int32, (TC_BLK, NUM_ANCHORS), 1)
        onehot = (iota == idc).astype(jnp.bfloat16)
        cg = jnp.dot(onehot, cent_ref[...],
                     preferred_element_type=jnp.float32)
        diff = z_ref[...] - cg
        dist2 = jnp.sum(diff * diff, axis=1, keepdims=True)
        h = jnp.maximum(jnp.sqrt(dist2) - MARGIN, 0.0)
        hs = jnp.sum(jnp.where(valid, h, 0.0))
        cnt = jnp.sum(valid.astype(jnp.float32))
        lane = lax.broadcasted_iota(jnp.int32, (1, 128), 1)
        out_ref[pl.ds(pl.program_id(0), 1), :] = (
            jnp.where(lane == 0, hs, 0.0) + jnp.where(lane == 1, cnt, 0.0))

    return pl.pallas_call(
        tc_body,
        grid=(grid,),
        in_specs=[
            pl.BlockSpec((TC_BLK, D), lambda i: (offset_blocks + i, 0)),
            pl.BlockSpec((TC_BLK, 1), lambda i: (offset_blocks + i, 0)),
            pl.BlockSpec((NUM_ANCHORS, D), lambda i: (0, 0)),
        ],
        out_specs=pl.BlockSpec((grid, 128), lambda i: (0, 0)),
        out_shape=jax.ShapeDtypeStruct((grid, 128), jnp.float32),
    )


SC_TOKENS = 16384


@jax.jit
def kernel(z_q, root_ids, centroids):
    b, s, d = z_q.shape
    total = b * s
    z2 = z_q.reshape(total, d)
    na = centroids.shape[0]
    cent_bf = centroids.astype(jnp.bfloat16)
    cb3 = lax.bitcast_convert_type(
        cent_bf.reshape(na, d // 32, 2, 16), jnp.uint16).astype(jnp.int32)
    cpack = (cb3[:, :, 0, :] | (cb3[:, :, 1, :] << 16)).reshape(na, d // 2)
    ids = root_ids.reshape(total)
    sc_parts = _make_sc_kernel(total)(z2, ids, cpack)
    hinge_total = jnp.sum(sc_parts[:, :16])
    count = jnp.sum(sc_parts[:, 16:])
    return jnp.where(count > 0, hinge_total / jnp.maximum(count, 1.0), 0.0)


# speedup vs baseline: 1.1110x; 1.1110x over previous
import functools

import jax
import jax.numpy as jnp
from jax import lax
from jax.experimental import pallas as pl
from jax.experimental.pallas import tpu as pltpu
from jax.experimental.pallas import tpu_sc as plsc

MARGIN = 0.5
NUM_ANCHORS = 2048
D = 256
NC = 2
NS = 16
NW = NC * NS
L = 16
DSL = D // L
DSL2 = D // (2 * L)


def _vsqrt(x):
    i = lax.bitcast_convert_type(x, jnp.int32)
    i = jnp.int32(0x5F3759DF) - lax.shift_right_arithmetic(i, 1)
    y = lax.bitcast_convert_type(i, jnp.float32)
    for _ in range(3):
        y = y * (1.5 - 0.5 * x * y * y)
    return x * y


def _make_sc_kernel(total_tokens):
    tpw = total_tokens // NW
    chunk = 64
    nbuf = 2
    nch = tpw // chunk
    mesh = plsc.VectorSubcoreMesh(
        core_axis_name="c", subcore_axis_name="s",
        num_cores=NC, num_subcores=NS)

    @functools.partial(
        pl.kernel,
        out_type=jax.ShapeDtypeStruct((NW, 2 * L), jnp.float32),
        mesh=mesh,
        compiler_params=pltpu.CompilerParams(needs_layout_passes=False),
        scratch_types=[
            pltpu.VMEM((tpw,), jnp.int32),
            pltpu.VMEM((tpw,), jnp.int32),
            pltpu.VMEM((nbuf, chunk, D), jnp.float32),
            pltpu.VMEM((nbuf, chunk, D // 2), jnp.int32),
            pltpu.VMEM((L * L,), jnp.float32),
            pltpu.VMEM((2 * L,), jnp.float32),
            pltpu.VMEM_SHARED((NUM_ANCHORS, D // 2), jnp.int32),
        ] + [pltpu.SemaphoreType.DMA] * (2 * nbuf),
    )
    def sc_kernel(z_hbm, ids_hbm, cent_hbm, out_hbm,
                  idx_raw, idx_cl, z_v, cent_v, m_v, acc_v, cent_sh,
                  *sems_flat):
        sid = lax.axis_index("s")
        wid = lax.axis_index("c") * NS + sid
        base = wid * tpw

        rps = NUM_ANCHORS // NS
        pltpu.sync_copy(cent_hbm.at[pl.ds(sid * rps, rps)],
                        cent_sh.at[pl.ds(sid * rps, rps)])

        pltpu.sync_copy(ids_hbm.at[pl.ds(base, tpw)], idx_raw)

        def clamp_body(j, _):
            v = idx_raw[pl.ds(j * L, L)]
            idx_cl[pl.ds(j * L, L)] = jnp.maximum(v, 0)
            return _
        lax.fori_loop(0, tpw // L, clamp_body, None)

        zero = jnp.zeros((L,), jnp.float32)
        acc_v[pl.ds(0, L)] = zero
        acc_v[pl.ds(L, L)] = zero

        sems = tuple((sems_flat[2 * b], sems_flat[2 * b + 1])
                     for b in range(nbuf))
        rows = lax.iota(jnp.int32, L)
        plsc.subcore_barrier()

        def copies(g, buf):
            sz, sc = sems[buf]
            return (
                pltpu.make_async_copy(
                    z_hbm.at[pl.ds(base + g * chunk, chunk)],
                    z_v.at[buf], sz),
                pltpu.make_async_copy(
                    cent_sh.at[idx_cl.at[pl.ds(g * chunk, chunk)]],
                    cent_v.at[buf], sc),
            )

        def issue(g, buf):
            for cp in copies(g, buf):
                cp.start()

        def process(g, buf, last):
            tok0 = g * chunk
            for cp in copies(g, buf):
                cp.wait()
            zb = z_v.at[buf]
            cb = cent_v.at[buf]

            def grp_body(q, _):
                t0 = q * L
                for i in range(L):
                    t = t0 + i
                    acc = None
                    for k in range(DSL2):
                        zlo = zb[t, pl.ds(k * 2 * L, L)]
                        zhi = zb[t, pl.ds(k * 2 * L + L, L)]
                        cw = plsc.bitcast(cb[t, pl.ds(k * L, L)],
                                          jnp.bfloat16)
                        clo, chi = plsc.unpack(
                            cw, format=plsc.PackFormat.INTERLEAVED)
                        dlo = zlo - clo.astype(jnp.float32)
                        dhi = zhi - chi.astype(jnp.float32)
                        sq = dlo * dlo + dhi * dhi
                        acc = sq if acc is None else acc + sq
                    m_v[pl.ds(i * L, L)] = acc
                x = plsc.load_gather(m_v, [rows * L])
                for j in range(1, L):
                    x = x + plsc.load_gather(m_v, [rows * L + j])
                raw = idx_raw[pl.ds(tok0 + t0, L)]
                valid = raw >= 0
                h = jnp.maximum(_vsqrt(x) - MARGIN, 0.0)
                plsc.addupdate(acc_v.at[pl.ds(0, L)],
                               jnp.where(valid, h, 0.0))
                plsc.addupdate(acc_v.at[pl.ds(L, L)],
                               jnp.where(valid, 1.0, 0.0))
                return _
            lax.fori_loop(0, chunk // L, grp_body, None)

            @pl.when(jnp.logical_not(last))
            def _issue_next():
                issue(g + nbuf, buf)

        for b in range(nbuf):
            issue(b, b)

        def ring_body(p, _):
            last = p >= nch // nbuf - 1
            for b in range(nbuf):
                process(nbuf * p + b, b, last)
            return _
        lax.fori_loop(0, nch // nbuf, ring_body, None)

        pltpu.sync_copy(acc_v, out_hbm.at[wid])

    return sc_kernel


TC_BLK = 512


def _make_tc_kernel(tc_tokens, offset_blocks):
    grid = tc_tokens // TC_BLK

    def tc_body(z_ref, ids_ref, cent_ref, out_ref):
        ids = ids_ref[...]
        valid = ids >= 0
        idc = jnp.maximum(ids, 0)
        iota = lax.broadcasted_iota(jnp.int32, (TC_BLK, NUM_ANCHORS), 1)
        onehot = (iota == idc).astype(jnp.bfloat16)
        cg = jnp.dot(onehot, cent_ref[...],
                     preferred_element_type=jnp.float32)
        diff = z_ref[...] - cg
        dist2 = jnp.sum(diff * diff, axis=1, keepdims=True)
        h = jnp.maximum(jnp.sqrt(dist2) - MARGIN, 0.0)
        hs = jnp.sum(jnp.where(valid, h, 0.0))
        cnt = jnp.sum(valid.astype(jnp.float32))
        lane = lax.broadcasted_iota(jnp.int32, (1, 128), 1)
        out_ref[pl.ds(pl.program_id(0), 1), :] = (
            jnp.where(lane == 0, hs, 0.0) + jnp.where(lane == 1, cnt, 0.0))

    return pl.pallas_call(
        tc_body,
        grid=(grid,),
        in_specs=[
            pl.BlockSpec((TC_BLK, D), lambda i: (offset_blocks + i, 0)),
            pl.BlockSpec((TC_BLK, 1), lambda i: (offset_blocks + i, 0)),
            pl.BlockSpec((NUM_ANCHORS, D), lambda i: (0, 0)),
        ],
        out_specs=pl.BlockSpec((grid, 128), lambda i: (0, 0)),
        out_shape=jax.ShapeDtypeStruct((grid, 128), jnp.float32),
    )


SC_TOKENS = 16384


@jax.jit
def kernel(z_q, root_ids, centroids):
    b, s, d = z_q.shape
    total = b * s
    z2 = z_q.reshape(total, d)
    na = centroids.shape[0]
    cent_bf = centroids.astype(jnp.bfloat16)
    cb3 = lax.bitcast_convert_type(
        cent_bf.reshape(na, d // 32, 2, 16), jnp.uint16).astype(jnp.int32)
    cpack = (cb3[:, :, 0, :] | (cb3[:, :, 1, :] << 16)).reshape(na, d // 2)
    ids = root_ids.reshape(total)
    sc_parts = _make_sc_kernel(total)(z2, ids, cpack)
    hinge_total = jnp.sum(sc_parts[:, :16])
    count = jnp.sum(sc_parts[:, 16:])
    return jnp.where(count > 0, hinge_total / jnp.maximum(count, 1.0), 0.0)


# speedup vs baseline: 1.1223x; 1.0101x over previous
import functools

import jax
import jax.numpy as jnp
from jax import lax
from jax.experimental import pallas as pl
from jax.experimental.pallas import tpu as pltpu
from jax.experimental.pallas import tpu_sc as plsc

MARGIN = 0.5
NUM_ANCHORS = 2048
D = 256
NC = 2
NS = 16
NW = NC * NS
L = 16
DSL = D // L
DSL2 = D // (2 * L)


def _vsqrt(x):
    i = lax.bitcast_convert_type(x, jnp.int32)
    i = jnp.int32(0x5F3759DF) - lax.shift_right_arithmetic(i, 1)
    y = lax.bitcast_convert_type(i, jnp.float32)
    for _ in range(3):
        y = y * (1.5 - 0.5 * x * y * y)
    return x * y


def _make_sc_kernel(total_tokens):
    tpw = total_tokens // NW
    chunk = 64
    nbuf = 2
    nch = tpw // chunk
    mesh = plsc.VectorSubcoreMesh(
        core_axis_name="c", subcore_axis_name="s",
        num_cores=NC, num_subcores=NS)

    @functools.partial(
        pl.kernel,
        out_type=jax.ShapeDtypeStruct((NW, 2 * L), jnp.float32),
        mesh=mesh,
        compiler_params=pltpu.CompilerParams(needs_layout_passes=False),
        scratch_types=[
            pltpu.VMEM((tpw,), jnp.int32),
            pltpu.VMEM((tpw,), jnp.int32),
            pltpu.VMEM((nbuf, chunk, D), jnp.float32),
            pltpu.VMEM((nbuf, chunk, D // 2), jnp.int32),
            pltpu.VMEM((L * L,), jnp.float32),
            pltpu.VMEM((2 * L,), jnp.float32),
        ] + [pltpu.SemaphoreType.DMA] * (2 * nbuf),
    )
    def sc_kernel(z_hbm, ids_hbm, cent_hbm, out_hbm,
                  idx_raw, idx_cl, z_v, cent_v, m_v, acc_v, *sems_flat):
        wid = lax.axis_index("c") * NS + lax.axis_index("s")
        base = wid * tpw

        pltpu.sync_copy(ids_hbm.at[pl.ds(base, tpw)], idx_raw)

        def clamp_body(j, _):
            v = idx_raw[pl.ds(j * L, L)]
            idx_cl[pl.ds(j * L, L)] = jnp.maximum(v, 0)
            return _
        lax.fori_loop(0, tpw // L, clamp_body, None)

        zero = jnp.zeros((L,), jnp.float32)
        acc_v[pl.ds(0, L)] = zero
        acc_v[pl.ds(L, L)] = zero

        sems = tuple((sems_flat[2 * b], sems_flat[2 * b + 1])
                     for b in range(nbuf))
        rows = lax.iota(jnp.int32, L)

        def copies(g, buf):
            sz, sc = sems[buf]
            return (
                pltpu.make_async_copy(
                    z_hbm.at[pl.ds(base + g * chunk, chunk)],
                    z_v.at[buf], sz),
                pltpu.make_async_copy(
                    cent_hbm.at[idx_cl.at[pl.ds(g * chunk, chunk)]],
                    cent_v.at[buf], sc),
            )

        def issue(g, buf):
            for cp in copies(g, buf):
                cp.start()

        def process(g, buf, last):
            tok0 = g * chunk
            for cp in copies(g, buf):
                cp.wait()
            zb = z_v.at[buf]
            cb = cent_v.at[buf]

            def grp_body(q, _):
                t0 = q * L
                for i in range(L):
                    t = t0 + i
                    acc = None
                    for k in range(DSL2):
                        zlo = zb[t, pl.ds(k * 2 * L, L)]
                        zhi = zb[t, pl.ds(k * 2 * L + L, L)]
                        cw = plsc.bitcast(cb[t, pl.ds(k * L, L)],
                                          jnp.bfloat16)
                        clo, chi = plsc.unpack(
                            cw, format=plsc.PackFormat.INTERLEAVED)
                        dlo = zlo - clo.astype(jnp.float32)
                        dhi = zhi - chi.astype(jnp.float32)
                        sq = dlo * dlo + dhi * dhi
                        acc = sq if acc is None else acc + sq
                    m_v[pl.ds(i * L, L)] = acc
                x = plsc.load_gather(m_v, [rows * L])
                for j in range(1, L):
                    x = x + plsc.load_gather(m_v, [rows * L + j])
                raw = idx_raw[pl.ds(tok0 + t0, L)]
                valid = raw >= 0
                h = jnp.maximum(_vsqrt(x) - MARGIN, 0.0)
                plsc.addupdate(acc_v.at[pl.ds(0, L)],
                               jnp.where(valid, h, 0.0))
                plsc.addupdate(acc_v.at[pl.ds(L, L)],
                               jnp.where(valid, 1.0, 0.0))
                return _
            lax.fori_loop(0, chunk // L, grp_body, None)

            @pl.when(jnp.logical_not(last))
            def _issue_next():
                issue(g + nbuf, buf)

        for b in range(nbuf):
            issue(b, b)

        def ring_body(p, _):
            last = p >= nch // nbuf - 1
            for b in range(nbuf):
                process(nbuf * p + b, b, last)
            return _
        lax.fori_loop(0, nch // nbuf, ring_body, None)

        pltpu.sync_copy(acc_v, out_hbm.at[wid])

    return sc_kernel


@jax.jit
def kernel(z_q, root_ids, centroids):
    b, s, d = z_q.shape
    total = b * s
    z2 = z_q.reshape(total, d)
    na = centroids.shape[0]
    cb3 = lax.bitcast_convert_type(
        centroids.astype(jnp.bfloat16).reshape(na, d // 32, 2, 16),
        jnp.uint16).astype(jnp.int32)
    cpack = (cb3[:, :, 0, :] | (cb3[:, :, 1, :] << 16)).reshape(na, d // 2)
    ids = root_ids.reshape(total)
    sc_parts = _make_sc_kernel(total)(z2, ids, cpack)
    hinge_total = jnp.sum(sc_parts[:, :16])
    count = jnp.sum(sc_parts[:, 16:])
    return jnp.where(count > 0, hinge_total / jnp.maximum(count, 1.0), 0.0)
